# Initial kernel scaffold; baseline (speedup 1.0000x reference)
import jax
import jax.numpy as jnp
from jax.experimental import pallas as pl


def kernel(data, ent_emb, rel_emb, conv_W, conv_b, fc_W, bn1_w, bn1_b, bn1_rm, bn1_rv, bn2_w, bn2_b, bn2_rm, bn2_rv):
    raise NotImplementedError("write your pallas kernel here")


# speedup vs baseline: 2.4505x; 2.4505x over previous
import functools

import jax
import jax.numpy as jnp
from jax.experimental import pallas as pl

_EPS = 1e-5
_BLK = 1000


def _tables_body(ent_ref, rel_ref, ab_ref, w0_ref, w1_ref, w2_ref,
                 sh_ref, sr_ref, st_ref):
    e = ent_ref[:, :]
    rl = rel_ref[:, :]
    acc_h = jnp.zeros(e.shape, jnp.float32)
    acc_t = jnp.zeros(e.shape, jnp.float32)
    acc_r = jnp.zeros(e.shape, jnp.float32)
    for ch in range(32):
        a = ab_ref[0, ch]
        b = ab_ref[1, ch]
        th = jnp.maximum(e * a + b, 0.0)
        tr = jnp.maximum(rl * a + b, 0.0)
        acc_h = acc_h + th * w0_ref[ch, :][None, :]
        acc_t = acc_t + th * w2_ref[ch, :][None, :]
        acc_r = acc_r + tr * w1_ref[ch, :][None, :]
    sh_ref[:, 0] = jnp.sum(acc_h, axis=1)
    st_ref[:, 0] = jnp.sum(acc_t, axis=1)
    sr_ref[:, 0] = jnp.sum(acc_r, axis=1)


@jax.jit
def _score_tables(ent_emb, rel_emb, ab, w0, w1, w2):
    n = ent_emb.shape[0]
    grid = (n // _BLK,)
    out_sd = jax.ShapeDtypeStruct((n, 1), jnp.float32)
    blk_row = pl.BlockSpec((_BLK, 128), lambda i: (i, 0))
    blk_full = pl.BlockSpec((32, 128), lambda i: (0, 0))
    blk_ab = pl.BlockSpec((2, 32), lambda i: (0, 0))
    blk_out = pl.BlockSpec((_BLK, 1), lambda i: (i, 0))
    return pl.pallas_call(
        _tables_body,
        grid=grid,
        in_specs=[blk_row, blk_row, blk_ab, blk_full, blk_full, blk_full],
        out_specs=[blk_out, blk_out, blk_out],
        out_shape=[out_sd, out_sd, out_sd],
    )(ent_emb, rel_emb, ab, w0, w1, w2)


def kernel(data, ent_emb, rel_emb, conv_W, conv_b, fc_W,
           bn1_w, bn1_b, bn1_rm, bn1_rv, bn2_w, bn2_b, bn2_rm, bn2_rv):
    n, dim = ent_emb.shape
    e_num = data.shape[0]
    h = data[:, 0]
    r = data[:, 1]
    t = data[:, 2]

    a1 = (bn1_w / jnp.sqrt(bn1_rv + _EPS))[0]
    c1 = (bn1_b - bn1_rm * (bn1_w / jnp.sqrt(bn1_rv + _EPS)))[0]
    s2 = bn2_w / jnp.sqrt(bn2_rv + _EPS)
    w = conv_W.reshape(-1)
    alpha = a1 * w * s2
    beta = (c1 * w + conv_b - bn2_rm) * s2 + bn2_b
    ab = jnp.stack([alpha, beta])
    fc3 = fc_W.reshape(32, dim, 3)
    w0 = fc3[:, :, 0]
    w1 = fc3[:, :, 1]
    w2 = fc3[:, :, 2]

    sh, sr, st = _score_tables(ent_emb, rel_emb, ab, w0, w1, w2)
    sh = sh[:, 0]
    sr = sr[:, 0]
    st = st[:, 0]
    score = sh[h] + sr[r] + st[t]

    diag = jnp.arange(n, dtype=h.dtype)
    rows = jnp.concatenate([h, diag])
    cols = jnp.concatenate([t, diag])
    vals = jnp.concatenate([score, jnp.ones((n,), jnp.float32)])
    key = rows * n + cols
    skey, svals = jax.lax.sort((key, vals), num_keys=1)
    m_tot = e_num + n
    is_start = jnp.concatenate(
        [jnp.ones((1,), bool), skey[1:] != skey[:-1]])
    uid = jnp.cumsum(is_start.astype(jnp.int32)) - 1
    coal = jnp.zeros((m_tot,), jnp.float32).at[uid].add(svals)
    urow = jnp.zeros((m_tot,), jnp.int32).at[uid].set(skey // n)
    ucol = jnp.zeros((m_tot,), jnp.int32).at[uid].set(skey % n)
    nuniq = uid[-1] + 1
    valid = jnp.arange(m_tot, dtype=jnp.int32) < nuniq

    shift = jnp.zeros((n,), jnp.float32).at[rows].add(jnp.abs(vals))
    wexp = jnp.where(valid, jnp.exp(coal - shift[urow]), 0.0)
    z = jnp.zeros((n,), jnp.float32).at[urow].add(wexp)
    num = jnp.zeros((n, dim), jnp.float32).at[urow].add(
        wexp[:, None] * ent_emb[ucol])
    return num / z[:, None]


# speedup vs baseline: 3.0557x; 1.2470x over previous
import functools

import jax
import jax.numpy as jnp
from jax import lax
from jax.experimental import pallas as pl
from jax.experimental.pallas import tpu as pltpu
from jax.experimental.pallas import tpu_sc as plsc

_EPS = 1e-5
_BLK = 1000

_N = 10000
_NPAD = 10240
_MP = 172032
_PW = 5376
_NCH = _PW // 128


def _agg_body(rows_h, cols_h, a_h, wm_h, shift_h, emb_h, zn_h, z2_h,
              zpart, numpart,
              rows_v, cols_v, a_v, wm_v, shg_v, w_v, g_v, z_s, num_s, sem):
    c = lax.axis_index("c")
    s = lax.axis_index("s")
    wid = s * 2 + c
    sl640 = pl.ds(s * 640, 640)
    pltpu.sync_copy(zn_h.at[sl640], z_s.at[sl640])
    pltpu.sync_copy(z2_h.at[sl640], num_s.at[sl640])
    pltpu.sync_copy(rows_h.at[wid], rows_v)
    pltpu.sync_copy(cols_h.at[wid], cols_v)
    pltpu.sync_copy(a_h.at[wid], a_v)
    pltpu.sync_copy(wm_h.at[wid], wm_v)
    plsc.subcore_barrier()

    def chunk(kk, carry):
        idx = rows_v.at[kk]
        cidx = cols_v.at[kk]
        pltpu.async_copy(shift_h.at[idx], shg_v, sem).wait()
        pltpu.async_copy(emb_h.at[cidx], g_v, sem).wait()
        for j in range(8):
            slj = pl.ds(j * 16, 16)
            w_v[slj] = wm_v[kk, slj] * jnp.exp(a_v[kk, slj] - shg_v[slj])

        for j16 in range(8):
            wvec = w_v[pl.ds(j16 * 16, 16)]
            for l in range(16):
                wl = wvec[l]
                j = j16 * 16 + l
                for q in range(8):
                    qs = pl.ds(q * 16, 16)
                    g_v[j, qs] = g_v[j, qs] * wl
        pltpu.sync_copy(w_v, z_s.at[idx], add=True)
        pltpu.sync_copy(g_v, num_s.at[idx], add=True)
        return carry

    lax.fori_loop(0, _NCH, chunk, 0)
    plsc.subcore_barrier()
    out_off = c * _NPAD + s * 640
    pltpu.sync_copy(z_s.at[sl640], zpart.at[pl.ds(out_off, 640)])
    pltpu.sync_copy(num_s.at[sl640], numpart.at[pl.ds(out_off, 640)])


@jax.jit
def _sc_softmax_agg(rows2, cols2, a2, wm2, shift, emb):
    zn = jnp.zeros((_NPAD,), jnp.float32)
    z2 = jnp.zeros((_NPAD, 128), jnp.float32)
    k = functools.partial(
        pl.kernel,
        out_type=[jax.ShapeDtypeStruct((2 * _NPAD,), jnp.float32),
                  jax.ShapeDtypeStruct((2 * _NPAD, 128), jnp.float32)],
        mesh=plsc.VectorSubcoreMesh(core_axis_name="c", subcore_axis_name="s"),
        scratch_types=[
            pltpu.VMEM((_NCH, 128), jnp.int32),
            pltpu.VMEM((_NCH, 128), jnp.int32),
            pltpu.VMEM((_NCH, 128), jnp.float32),
            pltpu.VMEM((_NCH, 128), jnp.float32),
            pltpu.VMEM((128,), jnp.float32),
            pltpu.VMEM((128,), jnp.float32),
            pltpu.VMEM((128, 128), jnp.float32),
            pltpu.VMEM_SHARED((_NPAD,), jnp.float32),
            pltpu.VMEM_SHARED((_NPAD, 128), jnp.float32),
            pltpu.SemaphoreType.DMA,
        ],
    )(_agg_body)
    zpart, numpart = k(rows2, cols2, a2, wm2, shift, emb, zn, z2)
    z = zpart[:_NPAD][:_N] + zpart[_NPAD:][:_N]
    num = numpart[:_NPAD][:_N] + numpart[_NPAD:][:_N]
    return z, num


def _tables_body(ent_ref, rel_ref, ab_ref, w0_ref, w1_ref, w2_ref,
                 sh_ref, sr_ref, st_ref):
    e = ent_ref[:, :]
    rl = rel_ref[:, :]
    acc_h = jnp.zeros(e.shape, jnp.float32)
    acc_t = jnp.zeros(e.shape, jnp.float32)
    acc_r = jnp.zeros(e.shape, jnp.float32)
    for ch in range(32):
        a = ab_ref[0, ch]
        b = ab_ref[1, ch]
        th = jnp.maximum(e * a + b, 0.0)
        tr = jnp.maximum(rl * a + b, 0.0)
        acc_h = acc_h + th * w0_ref[ch, :][None, :]
        acc_t = acc_t + th * w2_ref[ch, :][None, :]
        acc_r = acc_r + tr * w1_ref[ch, :][None, :]
    sh_ref[:, 0] = jnp.sum(acc_h, axis=1)
    st_ref[:, 0] = jnp.sum(acc_t, axis=1)
    sr_ref[:, 0] = jnp.sum(acc_r, axis=1)


@jax.jit
def _score_tables(ent_emb, rel_emb, ab, w0, w1, w2):
    n = ent_emb.shape[0]
    grid = (n // _BLK,)
    out_sd = jax.ShapeDtypeStruct((n, 1), jnp.float32)
    blk_row = pl.BlockSpec((_BLK, 128), lambda i: (i, 0))
    blk_full = pl.BlockSpec((32, 128), lambda i: (0, 0))
    blk_ab = pl.BlockSpec((2, 32), lambda i: (0, 0))
    blk_out = pl.BlockSpec((_BLK, 1), lambda i: (i, 0))
    return pl.pallas_call(
        _tables_body,
        grid=grid,
        in_specs=[blk_row, blk_row, blk_ab, blk_full, blk_full, blk_full],
        out_specs=[blk_out, blk_out, blk_out],
        out_shape=[out_sd, out_sd, out_sd],
    )(ent_emb, rel_emb, ab, w0, w1, w2)


def kernel(data, ent_emb, rel_emb, conv_W, conv_b, fc_W,
           bn1_w, bn1_b, bn1_rm, bn1_rv, bn2_w, bn2_b, bn2_rm, bn2_rv):
    n, dim = ent_emb.shape
    e_num = data.shape[0]
    h = data[:, 0]
    r = data[:, 1]
    t = data[:, 2]

    a1 = (bn1_w / jnp.sqrt(bn1_rv + _EPS))[0]
    c1 = (bn1_b - bn1_rm * (bn1_w / jnp.sqrt(bn1_rv + _EPS)))[0]
    s2 = bn2_w / jnp.sqrt(bn2_rv + _EPS)
    w = conv_W.reshape(-1)
    alpha = a1 * w * s2
    beta = (c1 * w + conv_b - bn2_rm) * s2 + bn2_b
    ab = jnp.stack([alpha, beta])
    fc3 = fc_W.reshape(32, dim, 3)
    w0 = fc3[:, :, 0]
    w1 = fc3[:, :, 1]
    w2 = fc3[:, :, 2]

    sh, sr, st = _score_tables(ent_emb, rel_emb, ab, w0, w1, w2)
    sh = sh[:, 0]
    sr = sr[:, 0]
    st = st[:, 0]
    score = sh[h] + sr[r] + st[t]

    diag = jnp.arange(n, dtype=h.dtype)
    rows = jnp.concatenate([h, diag])
    cols = jnp.concatenate([t, diag])
    vals = jnp.concatenate([score, jnp.ones((n,), jnp.float32)])
    key = rows * n + cols
    skey, svals = jax.lax.sort((key, vals), num_keys=1)
    m_tot = e_num + n
    is_start = jnp.concatenate(
        [jnp.ones((1,), bool), skey[1:] != skey[:-1]])
    uid = jnp.cumsum(is_start.astype(jnp.int32)) - 1
    coal = jnp.zeros((m_tot,), jnp.float32).at[uid].add(svals)
    urow = jnp.zeros((m_tot,), jnp.int32).at[uid].set(skey // n)
    ucol = jnp.zeros((m_tot,), jnp.int32).at[uid].set(skey % n)
    nuniq = uid[-1] + 1
    valid = jnp.arange(m_tot, dtype=jnp.int32) < nuniq

    shift = jnp.zeros((n,), jnp.float32).at[rows].add(jnp.abs(vals))

    pad = _MP - m_tot
    zi = jnp.zeros((pad,), jnp.int32)
    zf = jnp.zeros((pad,), jnp.float32)
    rows2 = jnp.concatenate([urow, zi]).reshape(32, _NCH, 128)
    cols2 = jnp.concatenate([ucol, zi]).reshape(32, _NCH, 128)
    a2 = jnp.concatenate([coal, zf]).reshape(32, _NCH, 128)
    wm2 = jnp.concatenate([valid.astype(jnp.float32), zf]).reshape(
        32, _NCH, 128)
    z, num = _sc_softmax_agg(rows2, cols2, a2, wm2, shift, ent_emb)
    return num / z[:, None]


# speedup vs baseline: 25.1384x; 8.2267x over previous
import functools

import jax
import jax.numpy as jnp
from jax import lax
from jax.experimental import pallas as pl
from jax.experimental.pallas import tpu as pltpu
from jax.experimental.pallas import tpu_sc as plsc

_EPS = 1e-5
_BLK = 1000

_N = 10000
_NPAD = 10240
_E = 160000
_M = 170000
_MP = 172032
_PW = 5376
_NCH = _PW // 128
_TBL = 100_012_032


def _tables_body(ent_ref, rel_ref, ab_ref, w0_ref, w1_ref, w2_ref,
                 sh_ref, sr_ref, st_ref):
    e = ent_ref[:, :]
    rl = rel_ref[:, :]
    acc_h = jnp.zeros(e.shape, jnp.float32)
    acc_t = jnp.zeros(e.shape, jnp.float32)
    acc_r = jnp.zeros(e.shape, jnp.float32)
    for ch in range(32):
        a = ab_ref[0, ch]
        b = ab_ref[1, ch]
        th = jnp.maximum(e * a + b, 0.0)
        tr = jnp.maximum(rl * a + b, 0.0)
        acc_h = acc_h + th * w0_ref[ch, :][None, :]
        acc_t = acc_t + th * w2_ref[ch, :][None, :]
        acc_r = acc_r + tr * w1_ref[ch, :][None, :]
    sh_ref[:, 0] = jnp.sum(acc_h, axis=1)
    st_ref[:, 0] = jnp.sum(acc_t, axis=1)
    sr_ref[:, 0] = jnp.sum(acc_r, axis=1)


@jax.jit
def _score_tables(ent_emb, rel_emb, ab, w0, w1, w2):
    n = ent_emb.shape[0]
    grid = (n // _BLK,)
    out_sd = jax.ShapeDtypeStruct((n, 1), jnp.float32)
    blk_row = pl.BlockSpec((_BLK, 128), lambda i: (i, 0))
    blk_full = pl.BlockSpec((32, 128), lambda i: (0, 0))
    blk_ab = pl.BlockSpec((2, 32), lambda i: (0, 0))
    blk_out = pl.BlockSpec((_BLK, 1), lambda i: (i, 0))
    return pl.pallas_call(
        _tables_body,
        grid=grid,
        in_specs=[blk_row, blk_row, blk_ab, blk_full, blk_full, blk_full],
        out_specs=[blk_out, blk_out, blk_out],
        out_shape=[out_sd, out_sd, out_sd],
    )(ent_emb, rel_emb, ab, w0, w1, w2)


def _ka_body(rows_h, rr_h, cols_h, sh_h, sr_h, st_h, zn_h,
             vals_o, keys_o, shpart_o, tbl_o,
             rows_v, rr_v, cols_v, vals_v, keys_v,
             g1_v, g2_v, g3_v, gi_v, av_v, sh_s, sem):
    c = lax.axis_index("c")
    s = lax.axis_index("s")
    wid = s * 2 + c
    sl640 = pl.ds(s * 640, 640)
    pltpu.sync_copy(zn_h.at[sl640], sh_s.at[sl640])
    pltpu.sync_copy(rows_h.at[wid], rows_v)
    pltpu.sync_copy(rr_h.at[wid], rr_v)
    pltpu.sync_copy(cols_h.at[wid], cols_v)
    plsc.subcore_barrier()

    def chunk(kk, carry):
        idx = rows_v.at[kk]
        pltpu.async_copy(sh_h.at[idx], g1_v, sem).wait()
        pltpu.async_copy(sr_h.at[rr_v.at[kk]], g2_v, sem).wait()
        pltpu.async_copy(st_h.at[cols_v.at[kk]], g3_v, sem).wait()
        gbase = (wid * _NCH + kk) * 128
        for j in range(8):
            slj = pl.ds(j * 16, 16)
            gi = gbase + j * 16 + lax.iota(jnp.int32, 16)
            val = g1_v[slj] + g2_v[slj] + g3_v[slj]
            val = jnp.where(gi < _E, val,
                            jnp.where(gi < _M, 1.0, 0.0))
            vals_v[kk, slj] = val
            av_v[slj] = jnp.abs(val)
            key = rows_v[kk, slj] * _N + cols_v[kk, slj]
            key = jnp.where(gi < _M, key, gi + (_TBL - _MP))
            keys_v[kk, slj] = key
            gi_v[slj] = gi
        pltpu.sync_copy(av_v, sh_s.at[idx], add=True)
        pltpu.sync_copy(gi_v, tbl_o.at[keys_v.at[kk]])
        return carry

    lax.fori_loop(0, _NCH, chunk, 0)
    pltpu.sync_copy(vals_v, vals_o.at[wid])
    pltpu.sync_copy(keys_v, keys_o.at[wid])
    plsc.subcore_barrier()
    pltpu.sync_copy(sh_s.at[sl640],
                    shpart_o.at[pl.ds(c * _NPAD + s * 640, 640)])


def _kb_body(keys_h, vals_h, tbl_h, za_h,
             win_o, accpart_o,
             keys_v, vals_v, win_v, acc_s, sem):
    c = lax.axis_index("c")
    s = lax.axis_index("s")
    wid = s * 2 + c
    slz = pl.ds(s * (_MP // 16), _MP // 16)
    pltpu.sync_copy(za_h.at[slz], acc_s.at[slz])
    pltpu.sync_copy(keys_h.at[wid], keys_v)
    pltpu.sync_copy(vals_h.at[wid], vals_v)
    plsc.subcore_barrier()

    def chunk(kk, carry):
        pltpu.async_copy(tbl_h.at[keys_v.at[kk]], win_v.at[kk], sem).wait()
        pltpu.sync_copy(vals_v.at[kk], acc_s.at[win_v.at[kk]], add=True)
        return carry

    lax.fori_loop(0, _NCH, chunk, 0)
    pltpu.sync_copy(win_v, win_o.at[wid])
    plsc.subcore_barrier()
    pltpu.sync_copy(acc_s.at[slz],
                    accpart_o.at[pl.ds(c * _MP + s * (_MP // 16),
                                       _MP // 16)])


@jax.jit
def _sc_coalesce(rows3, rr3, cols3, sh, sr, st):
    zn = jnp.zeros((_NPAD,), jnp.float32)
    za = jnp.zeros((_MP,), jnp.float32)
    i3 = jax.ShapeDtypeStruct((32, _NCH, 128), jnp.int32)
    f3 = jax.ShapeDtypeStruct((32, _NCH, 128), jnp.float32)
    ka = functools.partial(
        pl.kernel,
        out_type=[f3, i3,
                  jax.ShapeDtypeStruct((2 * _NPAD,), jnp.float32),
                  jax.ShapeDtypeStruct((_TBL,), jnp.int32)],
        mesh=plsc.VectorSubcoreMesh(core_axis_name="c", subcore_axis_name="s"),
        scratch_types=[
            pltpu.VMEM((_NCH, 128), jnp.int32),
            pltpu.VMEM((_NCH, 128), jnp.int32),
            pltpu.VMEM((_NCH, 128), jnp.int32),
            pltpu.VMEM((_NCH, 128), jnp.float32),
            pltpu.VMEM((_NCH, 128), jnp.int32),
            pltpu.VMEM((128,), jnp.float32),
            pltpu.VMEM((128,), jnp.float32),
            pltpu.VMEM((128,), jnp.float32),
            pltpu.VMEM((128,), jnp.int32),
            pltpu.VMEM((128,), jnp.float32),
            pltpu.VMEM_SHARED((_NPAD,), jnp.float32),
            pltpu.SemaphoreType.DMA,
        ],
    )(_ka_body)
    vals3, keys3, shpart, tbl = ka(rows3, rr3, cols3, sh, sr, st, zn)

    kb = functools.partial(
        pl.kernel,
        out_type=[i3, jax.ShapeDtypeStruct((2 * _MP,), jnp.float32)],
        mesh=plsc.VectorSubcoreMesh(core_axis_name="c", subcore_axis_name="s"),
        scratch_types=[
            pltpu.VMEM((_NCH, 128), jnp.int32),
            pltpu.VMEM((_NCH, 128), jnp.float32),
            pltpu.VMEM((_NCH, 128), jnp.int32),
            pltpu.VMEM_SHARED((_MP,), jnp.float32),
            pltpu.SemaphoreType.DMA,
        ],
    )(_kb_body)
    win3, accpart = kb(keys3, vals3, tbl, za)

    shift = shpart[:_NPAD][:_N] + shpart[_NPAD:][:_N]
    acc = accpart[:_MP] + accpart[_MP:]
    gi = jnp.arange(_MP, dtype=jnp.int32)
    wm = ((win3.reshape(-1) == gi) & (gi < _M)).astype(jnp.float32)
    return shift, acc, wm


def _agg_body(rows_h, cols_h, a_h, wm_h, shift_h, emb_h, zn_h, z2_h,
              zpart, numpart,
              rows_v, cols_v, a_v, wm_v, shg_v, w_v, g_v, z_s, num_s, sem):
    c = lax.axis_index("c")
    s = lax.axis_index("s")
    wid = s * 2 + c
    sl640 = pl.ds(s * 640, 640)
    pltpu.sync_copy(zn_h.at[sl640], z_s.at[sl640])
    pltpu.sync_copy(z2_h.at[sl640], num_s.at[sl640])
    pltpu.sync_copy(rows_h.at[wid], rows_v)
    pltpu.sync_copy(cols_h.at[wid], cols_v)
    pltpu.sync_copy(a_h.at[wid], a_v)
    pltpu.sync_copy(wm_h.at[wid], wm_v)
    plsc.subcore_barrier()

    def chunk(kk, carry):
        idx = rows_v.at[kk]
        cidx = cols_v.at[kk]
        pltpu.async_copy(shift_h.at[idx], shg_v, sem).wait()
        pltpu.async_copy(emb_h.at[cidx], g_v, sem).wait()
        for j in range(8):
            slj = pl.ds(j * 16, 16)
            w_v[slj] = wm_v[kk, slj] * jnp.exp(a_v[kk, slj] - shg_v[slj])

        for j16 in range(8):
            wvec = w_v[pl.ds(j16 * 16, 16)]
            for l in range(16):
                wl = wvec[l]
                j = j16 * 16 + l
                for q in range(8):
                    qs = pl.ds(q * 16, 16)
                    g_v[j, qs] = g_v[j, qs] * wl
        pltpu.sync_copy(w_v, z_s.at[idx], add=True)
        pltpu.sync_copy(g_v, num_s.at[idx], add=True)
        return carry

    lax.fori_loop(0, _NCH, chunk, 0)
    plsc.subcore_barrier()
    out_off = c * _NPAD + s * 640
    pltpu.sync_copy(z_s.at[sl640], zpart.at[pl.ds(out_off, 640)])
    pltpu.sync_copy(num_s.at[sl640], numpart.at[pl.ds(out_off, 640)])


@jax.jit
def _sc_softmax_agg(rows2, cols2, a2, wm2, shift, emb):
    zn = jnp.zeros((_NPAD,), jnp.float32)
    z2 = jnp.zeros((_NPAD, 128), jnp.float32)
    k = functools.partial(
        pl.kernel,
        out_type=[jax.ShapeDtypeStruct((2 * _NPAD,), jnp.float32),
                  jax.ShapeDtypeStruct((2 * _NPAD, 128), jnp.float32)],
        mesh=plsc.VectorSubcoreMesh(core_axis_name="c", subcore_axis_name="s"),
        scratch_types=[
            pltpu.VMEM((_NCH, 128), jnp.int32),
            pltpu.VMEM((_NCH, 128), jnp.int32),
            pltpu.VMEM((_NCH, 128), jnp.float32),
            pltpu.VMEM((_NCH, 128), jnp.float32),
            pltpu.VMEM((128,), jnp.float32),
            pltpu.VMEM((128,), jnp.float32),
            pltpu.VMEM((128, 128), jnp.float32),
            pltpu.VMEM_SHARED((_NPAD,), jnp.float32),
            pltpu.VMEM_SHARED((_NPAD, 128), jnp.float32),
            pltpu.SemaphoreType.DMA,
        ],
    )(_agg_body)
    zpart, numpart = k(rows2, cols2, a2, wm2, shift, emb, zn, z2)
    z = zpart[:_NPAD][:_N] + zpart[_NPAD:][:_N]
    num = numpart[:_NPAD][:_N] + numpart[_NPAD:][:_N]
    return z, num


def kernel(data, ent_emb, rel_emb, conv_W, conv_b, fc_W,
           bn1_w, bn1_b, bn1_rm, bn1_rv, bn2_w, bn2_b, bn2_rm, bn2_rv):
    n, dim = ent_emb.shape
    h = data[:, 0]
    r = data[:, 1]
    t = data[:, 2]

    a1 = (bn1_w / jnp.sqrt(bn1_rv + _EPS))[0]
    c1 = (bn1_b - bn1_rm * (bn1_w / jnp.sqrt(bn1_rv + _EPS)))[0]
    s2 = bn2_w / jnp.sqrt(bn2_rv + _EPS)
    w = conv_W.reshape(-1)
    alpha = a1 * w * s2
    beta = (c1 * w + conv_b - bn2_rm) * s2 + bn2_b
    ab = jnp.stack([alpha, beta])
    fc3 = fc_W.reshape(32, dim, 3)
    w0 = fc3[:, :, 0]
    w1 = fc3[:, :, 1]
    w2 = fc3[:, :, 2]

    sh, sr, st = _score_tables(ent_emb, rel_emb, ab, w0, w1, w2)
    sh = sh[:, 0]
    sr = sr[:, 0]
    st = st[:, 0]

    pad = _MP - _M
    diag = jnp.arange(n, dtype=jnp.int32)
    zi_pad = jnp.zeros((pad,), jnp.int32)
    zi_n = jnp.zeros((n + pad,), jnp.int32)
    rows3 = jnp.concatenate([h, diag, zi_pad]).reshape(32, _NCH, 128)
    rr3 = jnp.concatenate([r, zi_n]).reshape(32, _NCH, 128)
    cols3 = jnp.concatenate([t, diag, zi_pad]).reshape(32, _NCH, 128)

    shift, acc, wm = _sc_coalesce(rows3, rr3, cols3, sh, sr, st)
    a2 = acc.reshape(32, _NCH, 128)
    wm2 = wm.reshape(32, _NCH, 128)
    z, num = _sc_softmax_agg(rows3, cols3, a2, wm2, shift, ent_emb)
    return num / z[:, None]


# speedup vs baseline: 29.5111x; 1.1739x over previous
import functools

import jax
import jax.numpy as jnp
from jax import lax
from jax.experimental import pallas as pl
from jax.experimental.pallas import tpu as pltpu
from jax.experimental.pallas import tpu_sc as plsc

_EPS = 1e-5
_BLK = 1000

_N = 10000
_NPAD = 10240
_E = 160000
_M = 170000
_MP = 172032
_PW = 5376
_NCH = _PW // 128
_TBL = 100_012_032


def _tables_body(ent_ref, rel_ref, ab_ref, w0_ref, w1_ref, w2_ref,
                 sh_ref, sr_ref, st_ref):
    e = ent_ref[:, :]
    rl = rel_ref[:, :]
    acc_h = jnp.zeros(e.shape, jnp.float32)
    acc_t = jnp.zeros(e.shape, jnp.float32)
    acc_r = jnp.zeros(e.shape, jnp.float32)
    for ch in range(32):
        a = ab_ref[0, ch]
        b = ab_ref[1, ch]
        th = jnp.maximum(e * a + b, 0.0)
        tr = jnp.maximum(rl * a + b, 0.0)
        acc_h = acc_h + th * w0_ref[ch, :][None, :]
        acc_t = acc_t + th * w2_ref[ch, :][None, :]
        acc_r = acc_r + tr * w1_ref[ch, :][None, :]
    sh_ref[:, 0] = jnp.sum(acc_h, axis=1)
    st_ref[:, 0] = jnp.sum(acc_t, axis=1)
    sr_ref[:, 0] = jnp.sum(acc_r, axis=1)


@jax.jit
def _score_tables(ent_emb, rel_emb, ab, w0, w1, w2):
    n = ent_emb.shape[0]
    grid = (n // _BLK,)
    out_sd = jax.ShapeDtypeStruct((n, 1), jnp.float32)
    blk_row = pl.BlockSpec((_BLK, 128), lambda i: (i, 0))
    blk_full = pl.BlockSpec((32, 128), lambda i: (0, 0))
    blk_ab = pl.BlockSpec((2, 32), lambda i: (0, 0))
    blk_out = pl.BlockSpec((_BLK, 1), lambda i: (i, 0))
    return pl.pallas_call(
        _tables_body,
        grid=grid,
        in_specs=[blk_row, blk_row, blk_ab, blk_full, blk_full, blk_full],
        out_specs=[blk_out, blk_out, blk_out],
        out_shape=[out_sd, out_sd, out_sd],
    )(ent_emb, rel_emb, ab, w0, w1, w2)


def _ka_body(rows_h, rr_h, cols_h, sh_h, sr_h, st_h, zn_h,
             vals_o, keys_o, shpart_o, tbl_o,
             rows_v, rr_v, cols_v, vals_v, keys_v,
             g1_v, g2_v, g3_v, gi_v, av_v, sh_s, sem):
    c = lax.axis_index("c")
    s = lax.axis_index("s")
    wid = s * 2 + c
    sl640 = pl.ds(s * 640, 640)
    pltpu.sync_copy(zn_h.at[sl640], sh_s.at[sl640])
    pltpu.sync_copy(rows_h.at[wid], rows_v)
    pltpu.sync_copy(rr_h.at[wid], rr_v)
    pltpu.sync_copy(cols_h.at[wid], cols_v)
    plsc.subcore_barrier()

    def chunk(kk, carry):
        idx = rows_v.at[kk]
        cp1 = pltpu.async_copy(sh_h.at[idx], g1_v, sem)
        cp2 = pltpu.async_copy(sr_h.at[rr_v.at[kk]], g2_v, sem)
        cp3 = pltpu.async_copy(st_h.at[cols_v.at[kk]], g3_v, sem)
        cp1.wait()
        cp2.wait()
        cp3.wait()
        gbase = (wid * _NCH + kk) * 128
        for j in range(8):
            slj = pl.ds(j * 16, 16)
            gi = gbase + j * 16 + lax.iota(jnp.int32, 16)
            val = g1_v[slj] + g2_v[slj] + g3_v[slj]
            val = jnp.where(gi < _E, val,
                            jnp.where(gi < _M, 1.0, 0.0))
            vals_v[kk, slj] = val
            av_v[slj] = jnp.abs(val)
            key = rows_v[kk, slj] * _N + cols_v[kk, slj]
            key = jnp.where(gi < _M, key, gi + (_TBL - _MP))
            keys_v[kk, slj] = key
            gi_v[slj] = gi
        pltpu.sync_copy(av_v, sh_s.at[idx], add=True)
        pltpu.sync_copy(gi_v, tbl_o.at[keys_v.at[kk]])
        return carry

    lax.fori_loop(0, _NCH, chunk, 0)
    pltpu.sync_copy(vals_v, vals_o.at[wid])
    pltpu.sync_copy(keys_v, keys_o.at[wid])
    plsc.subcore_barrier()
    pltpu.sync_copy(sh_s.at[sl640],
                    shpart_o.at[pl.ds(c * _NPAD + s * 640, 640)])


def _kb_body(keys_h, vals_h, tbl_h, za_h,
             win_o, accpart_o,
             keys_v, vals_v, win_v, acc_s, sem, sem2):
    c = lax.axis_index("c")
    s = lax.axis_index("s")
    wid = s * 2 + c
    slz = pl.ds(s * (_MP // 16), _MP // 16)
    pltpu.sync_copy(za_h.at[slz], acc_s.at[slz])
    pltpu.sync_copy(keys_h.at[wid], keys_v)
    pltpu.sync_copy(vals_h.at[wid], vals_v)
    plsc.subcore_barrier()

    pltpu.async_copy(tbl_h.at[keys_v.at[0]], win_v.at[0], sem).wait()

    def chunk(kk, carry):
        @pl.when(kk < _NCH - 1)
        def _():
            pltpu.async_copy(tbl_h.at[keys_v.at[kk + 1]],
                             win_v.at[kk + 1], sem2)
        pltpu.sync_copy(vals_v.at[kk], acc_s.at[win_v.at[kk]], add=True)

        @pl.when(kk < _NCH - 1)
        def _():
            pltpu.make_async_copy(tbl_h.at[keys_v.at[kk + 1]],
                                  win_v.at[kk + 1], sem2).wait()
        return carry

    lax.fori_loop(0, _NCH, chunk, 0)
    pltpu.sync_copy(win_v, win_o.at[wid])
    plsc.subcore_barrier()
    pltpu.sync_copy(acc_s.at[slz],
                    accpart_o.at[pl.ds(c * _MP + s * (_MP // 16),
                                       _MP // 16)])


@jax.jit
def _sc_coalesce(rows3, rr3, cols3, sh, sr, st):
    zn = jnp.zeros((_NPAD,), jnp.float32)
    za = jnp.zeros((_MP,), jnp.float32)
    i3 = jax.ShapeDtypeStruct((32, _NCH, 128), jnp.int32)
    f3 = jax.ShapeDtypeStruct((32, _NCH, 128), jnp.float32)
    ka = functools.partial(
        pl.kernel,
        out_type=[f3, i3,
                  jax.ShapeDtypeStruct((2 * _NPAD,), jnp.float32),
                  jax.ShapeDtypeStruct((_TBL,), jnp.int32)],
        mesh=plsc.VectorSubcoreMesh(core_axis_name="c", subcore_axis_name="s"),
        scratch_types=[
            pltpu.VMEM((_NCH, 128), jnp.int32),
            pltpu.VMEM((_NCH, 128), jnp.int32),
            pltpu.VMEM((_NCH, 128), jnp.int32),
            pltpu.VMEM((_NCH, 128), jnp.float32),
            pltpu.VMEM((_NCH, 128), jnp.int32),
            pltpu.VMEM((128,), jnp.float32),
            pltpu.VMEM((128,), jnp.float32),
            pltpu.VMEM((128,), jnp.float32),
            pltpu.VMEM((128,), jnp.int32),
            pltpu.VMEM((128,), jnp.float32),
            pltpu.VMEM_SHARED((_NPAD,), jnp.float32),
            pltpu.SemaphoreType.DMA,
        ],
    )(_ka_body)
    vals3, keys3, shpart, tbl = ka(rows3, rr3, cols3, sh, sr, st, zn)

    kb = functools.partial(
        pl.kernel,
        out_type=[i3, jax.ShapeDtypeStruct((2 * _MP,), jnp.float32)],
        mesh=plsc.VectorSubcoreMesh(core_axis_name="c", subcore_axis_name="s"),
        scratch_types=[
            pltpu.VMEM((_NCH, 128), jnp.int32),
            pltpu.VMEM((_NCH, 128), jnp.float32),
            pltpu.VMEM((_NCH, 128), jnp.int32),
            pltpu.VMEM_SHARED((_MP,), jnp.float32),
            pltpu.SemaphoreType.DMA,
            pltpu.SemaphoreType.DMA,
        ],
    )(_kb_body)
    win3, accpart = kb(keys3, vals3, tbl, za)

    shift = shpart[:_NPAD][:_N] + shpart[_NPAD:][:_N]
    acc = accpart[:_MP] + accpart[_MP:]
    gi = jnp.arange(_MP, dtype=jnp.int32)
    wm = ((win3.reshape(-1) == gi) & (gi < _M)).astype(jnp.float32)
    return shift, acc, wm


def _agg_body(rows_h, cols_h, awm_h, shift_h, emb_h, zn_h, z2_h,
              zpart, numpart,
              rows_v, cols_v, awm_v, shg_v, w_v, g_v, z_s, num_s,
              sems, semg, seml):
    c = lax.axis_index("c")
    s = lax.axis_index("s")
    wid = s * 2 + c
    sl640 = pl.ds(s * 640, 640)
    pltpu.sync_copy(zn_h.at[sl640], z_s.at[sl640])
    pltpu.sync_copy(z2_h.at[sl640], num_s.at[sl640])
    pltpu.sync_copy(rows_h.at[wid], rows_v)
    pltpu.sync_copy(cols_h.at[wid], cols_v)
    plsc.subcore_barrier()

    pltpu.async_copy(shift_h.at[rows_v.at[0]], shg_v.at[0], sems)
    pltpu.async_copy(emb_h.at[cols_v.at[0]], g_v.at[0], semg)
    pltpu.async_copy(awm_h.at[wid, 0], awm_v.at[0], seml)

    def chunk(kk, carry):
        b = lax.rem(kk, 2)
        idx = rows_v.at[kk]
        pltpu.make_async_copy(shift_h.at[idx], shg_v.at[b], sems).wait()
        pltpu.make_async_copy(emb_h.at[cols_v.at[kk]], g_v.at[b],
                              semg).wait()
        pltpu.make_async_copy(awm_h.at[wid, kk], awm_v.at[b], seml).wait()

        @pl.when(kk < _NCH - 1)
        def _():
            nb = lax.rem(kk + 1, 2)
            pltpu.async_copy(shift_h.at[rows_v.at[kk + 1]],
                             shg_v.at[nb], sems)
            pltpu.async_copy(emb_h.at[cols_v.at[kk + 1]], g_v.at[nb], semg)
            pltpu.async_copy(awm_h.at[wid, kk + 1], awm_v.at[nb], seml)

        for j in range(8):
            slj = pl.ds(j * 16, 16)
            w_v[slj] = awm_v[b, 1, slj] * jnp.exp(
                awm_v[b, 0, slj] - shg_v[b, slj])

        for j16 in range(8):
            wvec = w_v[pl.ds(j16 * 16, 16)]
            for l in range(16):
                wl = wvec[l]
                j = j16 * 16 + l
                for q in range(8):
                    qs = pl.ds(q * 16, 16)
                    g_v[b, j, qs] = g_v[b, j, qs] * wl
        pltpu.sync_copy(w_v, z_s.at[idx], add=True)
        pltpu.sync_copy(g_v.at[b], num_s.at[idx], add=True)
        return carry

    lax.fori_loop(0, _NCH, chunk, 0)
    plsc.subcore_barrier()
    out_off = c * _NPAD + s * 640
    pltpu.sync_copy(z_s.at[sl640], zpart.at[pl.ds(out_off, 640)])
    pltpu.sync_copy(num_s.at[sl640], numpart.at[pl.ds(out_off, 640)])


@jax.jit
def _sc_softmax_agg(rows2, cols2, a2, wm2, shift, emb):
    zn = jnp.zeros((_NPAD,), jnp.float32)
    z2 = jnp.zeros((_NPAD, 128), jnp.float32)
    awm = jnp.stack([a2, wm2], axis=2)
    k = functools.partial(
        pl.kernel,
        out_type=[jax.ShapeDtypeStruct((2 * _NPAD,), jnp.float32),
                  jax.ShapeDtypeStruct((2 * _NPAD, 128), jnp.float32)],
        mesh=plsc.VectorSubcoreMesh(core_axis_name="c", subcore_axis_name="s"),
        scratch_types=[
            pltpu.VMEM((_NCH, 128), jnp.int32),
            pltpu.VMEM((_NCH, 128), jnp.int32),
            pltpu.VMEM((2, 2, 128), jnp.float32),
            pltpu.VMEM((2, 128), jnp.float32),
            pltpu.VMEM((128,), jnp.float32),
            pltpu.VMEM((2, 128, 128), jnp.float32),
            pltpu.VMEM_SHARED((_NPAD,), jnp.float32),
            pltpu.VMEM_SHARED((_NPAD, 128), jnp.float32),
            pltpu.SemaphoreType.DMA,
            pltpu.SemaphoreType.DMA,
            pltpu.SemaphoreType.DMA,
        ],
    )(_agg_body)
    zpart, numpart = k(rows2, cols2, awm, shift, emb, zn, z2)
    z = zpart[:_NPAD][:_N] + zpart[_NPAD:][:_N]
    num = numpart[:_NPAD][:_N] + numpart[_NPAD:][:_N]
    return z, num


def kernel(data, ent_emb, rel_emb, conv_W, conv_b, fc_W,
           bn1_w, bn1_b, bn1_rm, bn1_rv, bn2_w, bn2_b, bn2_rm, bn2_rv):
    n, dim = ent_emb.shape
    h = data[:, 0]
    r = data[:, 1]
    t = data[:, 2]

    a1 = (bn1_w / jnp.sqrt(bn1_rv + _EPS))[0]
    c1 = (bn1_b - bn1_rm * (bn1_w / jnp.sqrt(bn1_rv + _EPS)))[0]
    s2 = bn2_w / jnp.sqrt(bn2_rv + _EPS)
    w = conv_W.reshape(-1)
    alpha = a1 * w * s2
    beta = (c1 * w + conv_b - bn2_rm) * s2 + bn2_b
    ab = jnp.stack([alpha, beta])
    fc3 = fc_W.reshape(32, dim, 3)
    w0 = fc3[:, :, 0]
    w1 = fc3[:, :, 1]
    w2 = fc3[:, :, 2]

    sh, sr, st = _score_tables(ent_emb, rel_emb, ab, w0, w1, w2)
    sh = sh[:, 0]
    sr = sr[:, 0]
    st = st[:, 0]

    pad = _MP - _M
    diag = jnp.arange(n, dtype=jnp.int32)
    zi_pad = jnp.zeros((pad,), jnp.int32)
    zi_n = jnp.zeros((n + pad,), jnp.int32)
    rows3 = jnp.concatenate([h, diag, zi_pad]).reshape(32, _NCH, 128)
    rr3 = jnp.concatenate([r, zi_n]).reshape(32, _NCH, 128)
    cols3 = jnp.concatenate([t, diag, zi_pad]).reshape(32, _NCH, 128)

    shift, acc, wm = _sc_coalesce(rows3, rr3, cols3, sh, sr, st)
    a2 = acc.reshape(32, _NCH, 128)
    wm2 = wm.reshape(32, _NCH, 128)
    z, num = _sc_softmax_agg(rows3, cols3, a2, wm2, shift, ent_emb)
    return num / z[:, None]
